# speedup vs baseline: 3.5391x; 3.5391x over previous
import functools

import jax
import jax.numpy as jnp
from jax import lax
from jax.experimental import pallas as pl
from jax.experimental.pallas import tpu as pltpu
from jax.experimental.pallas import tpu_sc as plsc

B_TOK = 4
SEQ = 4096
D_MODEL = 1024
N_TOTAL = B_TOK * SEQ

NUM_CORES = 2
NUM_SUBCORES = 16
NW = NUM_CORES * NUM_SUBCORES
BPW = N_TOTAL // NW
CHUNK = 32
NCHUNK = BPW // CHUNK

_mesh = plsc.VectorSubcoreMesh(core_axis_name="c", subcore_axis_name="s")


@functools.partial(
    pl.kernel,
    out_type=jax.ShapeDtypeStruct((N_TOTAL, 1, D_MODEL), jnp.float32),
    mesh=_mesh,
    scratch_types=[
        pltpu.VMEM((NCHUNK, CHUNK), jnp.int32),
        pltpu.VMEM((CHUNK, 1, D_MODEL), jnp.float32),
        pltpu.VMEM((CHUNK, 1, D_MODEL), jnp.float32),
        pltpu.SemaphoreType.DMA,
        pltpu.SemaphoreType.DMA,
        pltpu.SemaphoreType.DMA,
        pltpu.SemaphoreType.DMA,
    ],
)
def _emb_lookup(idx_hbm, tbl_hbm, out_hbm, idx_v, buf0, buf1,
                gsem0, gsem1, ssem0, ssem1):
    wid = lax.axis_index("s") * NUM_CORES + lax.axis_index("c")
    base = wid * BPW

    pltpu.sync_copy(idx_hbm.at[wid], idx_v)

    bufs = (buf0, buf1)
    gsems = (gsem0, gsem1)
    ssems = (ssem0, ssem1)

    def gather(i):
        b = i % 2
        return pltpu.make_async_copy(
            tbl_hbm.at[idx_v.at[i]], bufs[b], gsems[b])

    def scatter(i):
        b = i % 2
        return pltpu.make_async_copy(
            bufs[b], out_hbm.at[pl.ds(base + i * CHUNK, CHUNK)], ssems[b])

    gather(0).start()
    for i in range(NCHUNK):
        gather(i).wait()
        scatter(i).start()
        if i + 1 < NCHUNK:
            if i >= 1:
                scatter(i - 1).wait()
            gather(i + 1).start()
    scatter(NCHUNK - 2).wait()
    scatter(NCHUNK - 1).wait()


def kernel(x, w_e):
    idx = x.reshape(NW, NCHUNK, CHUNK)
    out = _emb_lookup(idx, w_e)
    return out.reshape(B_TOK, SEQ, D_MODEL)


# speedup vs baseline: 3.6286x; 1.0253x over previous
import functools

import jax
import jax.numpy as jnp
from jax import lax
from jax.experimental import pallas as pl
from jax.experimental.pallas import tpu as pltpu
from jax.experimental.pallas import tpu_sc as plsc

B_TOK = 4
SEQ = 4096
D_MODEL = 1024
N_TOTAL = B_TOK * SEQ

NUM_CORES = 2
NUM_SUBCORES = 16
NW = NUM_CORES * NUM_SUBCORES
BPW = N_TOTAL // NW
CHUNK = 32
NCHUNK = BPW // CHUNK

_mesh = plsc.VectorSubcoreMesh(core_axis_name="c", subcore_axis_name="s")


@functools.partial(
    pl.kernel,
    out_type=jax.ShapeDtypeStruct((N_TOTAL, 1, D_MODEL), jnp.float32),
    mesh=_mesh,
    scratch_types=[
        pltpu.VMEM((NCHUNK, CHUNK), jnp.int32),
        pltpu.VMEM((CHUNK, 1, D_MODEL), jnp.float32),
        pltpu.VMEM((CHUNK, 1, D_MODEL), jnp.float32),
        pltpu.VMEM((CHUNK, 1, D_MODEL), jnp.float32),
        pltpu.SemaphoreType.DMA,
        pltpu.SemaphoreType.DMA,
        pltpu.SemaphoreType.DMA,
        pltpu.SemaphoreType.DMA,
        pltpu.SemaphoreType.DMA,
        pltpu.SemaphoreType.DMA,
    ],
)
def _emb_lookup(idx_hbm, tbl_hbm, out_hbm, idx_v, buf0, buf1, buf2,
                gsem0, gsem1, gsem2, ssem0, ssem1, ssem2):
    wid = lax.axis_index("s") * NUM_CORES + lax.axis_index("c")
    base = wid * BPW

    pltpu.sync_copy(idx_hbm.at[wid], idx_v)

    bufs = (buf0, buf1, buf2)
    gsems = (gsem0, gsem1, gsem2)
    ssems = (ssem0, ssem1, ssem2)
    NBUF = 3

    def gather(i):
        b = i % NBUF
        return pltpu.make_async_copy(
            tbl_hbm.at[idx_v.at[i]], bufs[b], gsems[b])

    def scatter(i):
        b = i % NBUF
        return pltpu.make_async_copy(
            bufs[b], out_hbm.at[pl.ds(base + i * CHUNK, CHUNK)], ssems[b])

    gather(0).start()
    gather(1).start()
    for i in range(NCHUNK):
        gather(i).wait()
        scatter(i).start()
        j = i + 2
        if j < NCHUNK:
            if j >= NBUF:
                scatter(j - NBUF).wait()
            gather(j).start()
    for i in range(NCHUNK - NBUF, NCHUNK):
        scatter(i).wait()


def kernel(x, w_e):
    idx = x.reshape(NW, NCHUNK, CHUNK)
    out = _emb_lookup(idx, w_e)
    return out.reshape(B_TOK, SEQ, D_MODEL)


# speedup vs baseline: 5.9895x; 1.6507x over previous
import functools

import jax
import jax.numpy as jnp
from jax import lax
from jax.experimental import pallas as pl
from jax.experimental.pallas import tpu as pltpu
from jax.experimental.pallas import tpu_sc as plsc

N_VOCAB = 100000
B_TOK = 4
SEQ = 4096
D_MODEL = 1024
N_TOTAL = B_TOK * SEQ
FRAG = 128
FPR = D_MODEL // FRAG

NUM_CORES = 2
NUM_SUBCORES = 16
NW = NUM_CORES * NUM_SUBCORES
BPW = N_TOTAL // NW
NFRAG = BPW * FPR
FCHUNK = 128
NCHUNK = NFRAG // FCHUNK
NPAIR = BPW // 16

_mesh = plsc.VectorSubcoreMesh(core_axis_name="c", subcore_axis_name="s")


@functools.partial(
    pl.kernel,
    out_type=jax.ShapeDtypeStruct((N_TOTAL * FPR, FRAG), jnp.float32),
    mesh=_mesh,
    compiler_params=pltpu.CompilerParams(
        needs_layout_passes=False, skip_device_barrier=True),
    scratch_types=[
        pltpu.VMEM((BPW // 128, 128), jnp.int32),
        pltpu.VMEM((NFRAG,), jnp.int32),
        pltpu.VMEM((FCHUNK, FRAG), jnp.float32),
        pltpu.VMEM((FCHUNK, FRAG), jnp.float32),
        pltpu.VMEM((FCHUNK, FRAG), jnp.float32),
        pltpu.VMEM((FCHUNK, FRAG), jnp.float32),
        pltpu.VMEM((FCHUNK, FRAG), jnp.float32),
        pltpu.VMEM((FCHUNK, FRAG), jnp.float32),
        pltpu.SemaphoreType.DMA,
        pltpu.SemaphoreType.DMA,
        pltpu.SemaphoreType.DMA,
        pltpu.SemaphoreType.DMA,
        pltpu.SemaphoreType.DMA,
        pltpu.SemaphoreType.DMA,
        pltpu.SemaphoreType.DMA,
        pltpu.SemaphoreType.DMA,
        pltpu.SemaphoreType.DMA,
        pltpu.SemaphoreType.DMA,
        pltpu.SemaphoreType.DMA,
        pltpu.SemaphoreType.DMA,
    ],
)
def _emb_lookup(idx_hbm, tbl_hbm, out_hbm, idx_v, fidx_v,
                buf0, buf1, buf2, buf3, buf4, buf5,
                gsem0, gsem1, gsem2, gsem3, gsem4, gsem5,
                ssem0, ssem1, ssem2, ssem3, ssem4, ssem5):
    wid = lax.axis_index("s") * NUM_CORES + lax.axis_index("c")
    fbase = wid * NFRAG

    pltpu.sync_copy(idx_hbm.at[pl.ds(wid * (BPW // 128), BPW // 128)], idx_v)

    lane = lax.iota(jnp.int32, 16)
    colbase = lane % 8 + jnp.where(lane >= 8, 64, 0)

    def build_pair(p, carry=0):
        ids = idx_v[p // 8, pl.ds(p % 8 * 16, 16)]
        pos = p * FCHUNK + colbase
        for t in range(FPR):
            plsc.store_scatter(fidx_v, [pos + t * 8], ids * FPR + t)
        return carry

    bufs = (buf0, buf1, buf2, buf3, buf4, buf5)
    gsems = (gsem0, gsem1, gsem2, gsem3, gsem4, gsem5)
    ssems = (ssem0, ssem1, ssem2, ssem3, ssem4, ssem5)
    NBUF = 6
    AHEAD = 4

    def gather(i):
        b = i % NBUF
        return pltpu.make_async_copy(
            tbl_hbm.at[fidx_v.at[pl.ds(i * FCHUNK, FCHUNK)]], bufs[b], gsems[b])

    def scatter(i):
        b = i % NBUF
        return pltpu.make_async_copy(
            bufs[b], out_hbm.at[pl.ds(fbase + i * FCHUNK, FCHUNK)], ssems[b])

    for i in range(AHEAD):
        build_pair(i)
        gather(i).start()
    lax.fori_loop(AHEAD, NPAIR, build_pair, 0)
    for i in range(NCHUNK):
        gather(i).wait()
        scatter(i).start()
        j = i + AHEAD
        if j < NCHUNK:
            if j >= NBUF:
                scatter(j - NBUF).wait()
            gather(j).start()
    for i in range(NCHUNK - NBUF, NCHUNK):
        scatter(i).wait()


def kernel(x, w_e):
    idx = x.reshape(128, 128)
    frags = _emb_lookup(idx, w_e.reshape(N_VOCAB * FPR, FRAG))
    return (frags.reshape(N_TOTAL // 8, FPR, 8, FRAG)
                 .transpose(0, 2, 1, 3)
                 .reshape(B_TOK, SEQ, D_MODEL))
